# baseline (speedup 1.0000x reference)
import functools

import jax
import jax.numpy as jnp
from jax import lax
from jax.experimental import pallas as pl
from jax.experimental.pallas import tpu as pltpu
from jax.experimental.pallas import tpu_sc as plsc


def _iota16():
    return jnp.arange(16, dtype=jnp.int32)


def _make_pack_kernel(vocab, dim, mesh, nw, nc):
    n_full = vocab // 128
    tiles_per_w = (n_full + nw - 1) // nw

    @functools.partial(
        pl.kernel,
        out_type=jax.ShapeDtypeStruct((vocab // 2, 128), jnp.float32),
        mesh=mesh,
        scratch_types=[
            pltpu.VMEM((2, 64, 128), jnp.float32),
            pltpu.VMEM((2, 64, 128), jnp.float32),
            pltpu.VMEM((32, 128), jnp.float32),
            pltpu.SemaphoreType.DMA,
            pltpu.SemaphoreType.DMA,
            pltpu.SemaphoreType.DMA,
            pltpu.SemaphoreType.DMA,
        ],
        compiler_params=pltpu.CompilerParams(
            use_tc_tiling_on_sc=True, needs_layout_passes=False, disable_bounds_checks=True
        ),
    )
    def pack_table(tview_hbm, tail_hbm, tpack_hbm, ibuf, obuf, tailbuf, si0, si1, so0, so1):
        wid = lax.axis_index("s") * nc + lax.axis_index("c")
        n_valid = jnp.minimum(tiles_per_w, jnp.maximum(n_full - wid * tiles_per_w, 0))
        sem_i = (si0, si1)
        sem_o = (so0, so1)
        iota = _iota16()

        def fire_read(k, sel):
            t = wid * tiles_per_w + k
            for g in range(8):
                pltpu.async_copy(
                    tview_hbm.at[pl.ds(8 * g, 8), pl.ds(t * 128, 128)],
                    ibuf.at[sel, pl.ds(8 * g, 8)],
                    sem_i[sel],
                )

        def wait_read(k, sel):
            t = wid * tiles_per_w + k
            for g in range(8):
                pltpu.make_async_copy(
                    tview_hbm.at[pl.ds(8 * g, 8), pl.ds(t * 128, 128)],
                    ibuf.at[sel, pl.ds(8 * g, 8)],
                    sem_i[sel],
                ).wait()

        def fire_store(k, sel):
            t = wid * tiles_per_w + k
            pltpu.async_copy(
                obuf.at[sel],
                tpack_hbm.at[pl.ds(t * 64, 64)],
                sem_o[sel],
            )

        def wait_store_bytes(sel):
            pltpu.make_async_copy(
                obuf.at[sel],
                tpack_hbm.at[pl.ds(0, 64)],
                sem_o[sel],
            ).wait()

        def transpose_tile(sel):
            for jj in range(8):
                row = jnp.full((16,), (jj * 16) % 64, jnp.int32) + iota
                cb = jj // 4
                for q in range(64):
                    col = jnp.full((16,), 2 * q + cb, jnp.int32)
                    vec = plsc.load_gather(ibuf.at[sel], [row, col])
                    obuf[sel, q, pl.ds(jj * 16, 16)] = vec

        fire_read(0, 0)

        @pl.loop(0, tiles_per_w + 1, step=2)
        def _tile(k2):
            for sel in (0, 1):
                k = k2 + sel
                nxt = k + 1

                @pl.when(nxt < n_valid)
                def _fire():
                    fire_read(nxt, 1 - sel)

                @pl.when(k < n_valid)
                def _proc():
                    wait_read(k, sel)

                    @pl.when(k >= 2)
                    def _ws():
                        wait_store_bytes(sel)

                    transpose_tile(sel)
                    fire_store(k, sel)

        wait_store_bytes(0)
        wait_store_bytes(1)

        @pl.when(wid == nw - 1)
        def _tail():
            pltpu.sync_copy(tail_hbm, tailbuf)
            pltpu.sync_copy(tailbuf, tpack_hbm.at[pl.ds(n_full * 64, 32)])

    return pack_table


def _make_gather_kernel(bsz, hist, dim, mesh, nw, nc):
    n_items = hist * (bsz // 128)
    items_per_w = n_items // nw
    assert n_items == items_per_w * nw
    c_per_h = bsz // 128

    @functools.partial(
        pl.kernel,
        out_type=jax.ShapeDtypeStruct((hist, 8, c_per_h, 8, 128), jnp.float32),
        mesh=mesh,
        scratch_types=[
            pltpu.VMEM((items_per_w, 128), jnp.int32),
            pltpu.VMEM((2, 128), jnp.int32),
            pltpu.VMEM((2, 128, 128), jnp.float32),
            pltpu.VMEM((2, 8, 8, 128), jnp.float32),
            pltpu.SemaphoreType.DMA,
            pltpu.SemaphoreType.DMA,
            pltpu.SemaphoreType.DMA,
            pltpu.SemaphoreType.DMA,
        ],
        compiler_params=pltpu.CompilerParams(
            use_tc_tiling_on_sc=False, needs_layout_passes=False, disable_bounds_checks=True
        ),
    )
    def gather_rows(tpack_hbm, idx3_hbm, out5_hbm, idx_all, prow, grows, obuf,
                    sg0, sg1, so0, so1):
        wid = lax.axis_index("s") * nc + lax.axis_index("c")
        sem_g = (sg0, sg1)
        sem_o = (so0, so1)
        iota = _iota16()

        pltpu.sync_copy(idx3_hbm.at[pl.ds(wid * items_per_w, items_per_w)], idx_all)

        def fire_gather(k, sel):
            for lc in range(8):
                v = idx_all[k, pl.ds(lc * 16, 16)]
                prow[sel, pl.ds(lc * 16, 16)] = v >> 1
            pltpu.async_copy(tpack_hbm.at[prow.at[sel]], grows.at[sel], sem_g[sel])

        def wait_gather(sel):
            pltpu.make_async_copy(
                tpack_hbm.at[prow.at[sel]], grows.at[sel], sem_g[sel]
            ).wait()

        def transpose_out(k, sel):
            for lc in range(8):
                v = idx_all[k, pl.ds(lc * 16, 16)]
                hoff = (v & 1) << 6
                row = jnp.full((16,), lc * 16, jnp.int32) + iota
                for g in range(8):
                    for s in range(8):
                        col = hoff + (8 * g + s)
                        vec = plsc.load_gather(grows.at[sel], [row, col])
                        obuf[sel, g, s, pl.ds(lc * 16, 16)] = vec

        def fire_store(k, sel):
            m = wid * items_per_w + k
            h = m // c_per_h
            c = m % c_per_h
            pltpu.async_copy(obuf.at[sel], out5_hbm.at[h, :, c], sem_o[sel])

        def wait_store_bytes(sel):
            pltpu.make_async_copy(
                obuf.at[sel], out5_hbm.at[0, :, 0], sem_o[sel]
            ).wait()

        fire_gather(0, 0)

        @pl.loop(0, items_per_w, step=2)
        def _item(k2):
            for sel in (0, 1):
                k = k2 + sel
                nxt = k + 1

                @pl.when(nxt < items_per_w)
                def _fire():
                    fire_gather(nxt, 1 - sel)

                wait_gather(sel)

                @pl.when(k >= 2)
                def _ws():
                    wait_store_bytes(sel)

                transpose_out(k, sel)
                fire_store(k, sel)

        wait_store_bytes(0)
        wait_store_bytes(1)

    return gather_rows


def kernel(indices, emb_table):
    bsz, hist = indices.shape
    vocab, dim = emb_table.shape
    assert dim == 64 and vocab % 128 == 64 and bsz % 128 == 0

    mesh = plsc.VectorSubcoreMesh(core_axis_name="c", subcore_axis_name="s")
    nc, ns = mesh.num_cores, mesh.num_subcores
    nw = nc * ns

    tview = emb_table.T
    n_full = vocab // 128
    tail = emb_table[n_full * 128:].reshape(32, 128)
    idx3 = indices.T.astype(jnp.int32).reshape(hist * (bsz // 128), 128)

    pack_table = _make_pack_kernel(vocab, dim, mesh, nw, nc)
    gather_rows = _make_gather_kernel(bsz, hist, dim, mesh, nw, nc)

    tpack = pack_table(tview, tail)
    out5 = gather_rows(tpack, idx3)
    return out5.transpose(2, 4, 0, 1, 3).reshape(bsz, hist, dim)


# speedup vs baseline: 1.7219x; 1.7219x over previous
import functools

import jax
import jax.numpy as jnp
from jax import lax
from jax.experimental import pallas as pl
from jax.experimental.pallas import tpu as pltpu
from jax.experimental.pallas import tpu_sc as plsc


def _iota16():
    return jnp.arange(16, dtype=jnp.int32)


def _make_pack_kernel(vocab, dim, mesh, nw, nc):
    n_full = vocab // 128
    tiles_per_w = (n_full + nw - 1) // nw

    @functools.partial(
        pl.kernel,
        out_type=jax.ShapeDtypeStruct((vocab // 2, 128), jnp.float32),
        mesh=mesh,
        scratch_types=[
            pltpu.VMEM((2, 64, 128), jnp.float32),
            pltpu.VMEM((2, 64, 128), jnp.float32),
            pltpu.VMEM((32, 128), jnp.float32),
            pltpu.SemaphoreType.DMA,
            pltpu.SemaphoreType.DMA,
            pltpu.SemaphoreType.DMA,
            pltpu.SemaphoreType.DMA,
        ],
        compiler_params=pltpu.CompilerParams(
            use_tc_tiling_on_sc=True, needs_layout_passes=False, disable_bounds_checks=True
        ),
    )
    def pack_table(tview_hbm, tail_hbm, tpack_hbm, ibuf, obuf, tailbuf, si0, si1, so0, so1):
        wid = lax.axis_index("s") * nc + lax.axis_index("c")
        n_valid = jnp.minimum(tiles_per_w, jnp.maximum(n_full - wid * tiles_per_w, 0))
        sem_i = (si0, si1)
        sem_o = (so0, so1)
        iota = _iota16()

        def fire_read(k, sel):
            t = wid * tiles_per_w + k
            for g in range(8):
                pltpu.async_copy(
                    tview_hbm.at[pl.ds(8 * g, 8), pl.ds(t * 128, 128)],
                    ibuf.at[sel, pl.ds(8 * g, 8)],
                    sem_i[sel],
                )

        def wait_read(k, sel):
            t = wid * tiles_per_w + k
            for g in range(8):
                pltpu.make_async_copy(
                    tview_hbm.at[pl.ds(8 * g, 8), pl.ds(t * 128, 128)],
                    ibuf.at[sel, pl.ds(8 * g, 8)],
                    sem_i[sel],
                ).wait()

        def fire_store(k, sel):
            t = wid * tiles_per_w + k
            pltpu.async_copy(
                obuf.at[sel],
                tpack_hbm.at[pl.ds(t * 64, 64)],
                sem_o[sel],
            )

        def wait_store_bytes(sel):
            pltpu.make_async_copy(
                obuf.at[sel],
                tpack_hbm.at[pl.ds(0, 64)],
                sem_o[sel],
            ).wait()

        def transpose_tile(sel):
            for jj in range(8):
                row = jnp.full((16,), (jj * 16) % 64, jnp.int32) + iota
                cb = jj // 4
                for q0 in range(0, 64, 8):
                    vecs = [
                        plsc.load_gather(
                            ibuf.at[sel],
                            [row, jnp.full((16,), 2 * (q0 + i) + cb, jnp.int32)],
                        )
                        for i in range(8)
                    ]
                    for i, vec in enumerate(vecs):
                        obuf[sel, q0 + i, pl.ds(jj * 16, 16)] = vec

        fire_read(0, 0)

        @pl.loop(0, tiles_per_w + 1, step=2)
        def _tile(k2):
            for sel in (0, 1):
                k = k2 + sel
                nxt = k + 1

                @pl.when(nxt < n_valid)
                def _fire():
                    fire_read(nxt, 1 - sel)

                @pl.when(k < n_valid)
                def _proc():
                    wait_read(k, sel)

                    @pl.when(k >= 2)
                    def _ws():
                        wait_store_bytes(sel)

                    transpose_tile(sel)
                    fire_store(k, sel)

        wait_store_bytes(0)
        wait_store_bytes(1)

        @pl.when(wid == nw - 1)
        def _tail():
            pltpu.sync_copy(tail_hbm, tailbuf)
            pltpu.sync_copy(tailbuf, tpack_hbm.at[pl.ds(n_full * 64, 32)])

    return pack_table


def _make_gather_kernel(bsz, hist, dim, mesh, nw, nc):
    n_items = hist * (bsz // 128)
    items_per_w = n_items // nw
    assert n_items == items_per_w * nw
    c_per_h = bsz // 128

    @functools.partial(
        pl.kernel,
        out_type=jax.ShapeDtypeStruct((hist, 8, c_per_h, 8, 128), jnp.float32),
        mesh=mesh,
        scratch_types=[
            pltpu.VMEM((items_per_w, 128), jnp.int32),
            pltpu.VMEM((2, 128), jnp.int32),
            pltpu.VMEM((2, 128, 128), jnp.float32),
            pltpu.VMEM((2, 8, 8, 128), jnp.float32),
            pltpu.SemaphoreType.DMA,
            pltpu.SemaphoreType.DMA,
            pltpu.SemaphoreType.DMA,
            pltpu.SemaphoreType.DMA,
        ],
        compiler_params=pltpu.CompilerParams(
            use_tc_tiling_on_sc=False, needs_layout_passes=False, disable_bounds_checks=True
        ),
    )
    def gather_rows(tpack_hbm, idx3_hbm, out5_hbm, idx_all, prow, grows, obuf,
                    sg0, sg1, so0, so1):
        wid = lax.axis_index("s") * nc + lax.axis_index("c")
        sem_g = (sg0, sg1)
        sem_o = (so0, so1)
        iota = _iota16()

        pltpu.sync_copy(idx3_hbm.at[pl.ds(wid * items_per_w, items_per_w)], idx_all)

        def fire_gather(k, sel):
            for lc in range(8):
                v = idx_all[k, pl.ds(lc * 16, 16)]
                prow[sel, pl.ds(lc * 16, 16)] = v >> 1
            pltpu.async_copy(tpack_hbm.at[prow.at[sel]], grows.at[sel], sem_g[sel])

        def wait_gather(sel):
            pltpu.make_async_copy(
                tpack_hbm.at[prow.at[sel]], grows.at[sel], sem_g[sel]
            ).wait()

        def transpose_out(k, sel):
            for lc in range(8):
                v = idx_all[k, pl.ds(lc * 16, 16)]
                hoff = (v & 1) << 6
                row = jnp.full((16,), lc * 16, jnp.int32) + iota
                for d0 in range(0, 64, 8):
                    vecs = [
                        plsc.load_gather(grows.at[sel], [row, hoff + (d0 + i)])
                        for i in range(8)
                    ]
                    for i, vec in enumerate(vecs):
                        d = d0 + i
                        obuf[sel, d // 8, d % 8, pl.ds(lc * 16, 16)] = vec

        def fire_store(k, sel):
            m = wid * items_per_w + k
            h = m // c_per_h
            c = m % c_per_h
            pltpu.async_copy(obuf.at[sel], out5_hbm.at[h, :, c], sem_o[sel])

        def wait_store_bytes(sel):
            pltpu.make_async_copy(
                obuf.at[sel], out5_hbm.at[0, :, 0], sem_o[sel]
            ).wait()

        fire_gather(0, 0)

        @pl.loop(0, items_per_w, step=2)
        def _item(k2):
            for sel in (0, 1):
                k = k2 + sel
                nxt = k + 1

                @pl.when(nxt < items_per_w)
                def _fire():
                    fire_gather(nxt, 1 - sel)

                wait_gather(sel)

                @pl.when(k >= 2)
                def _ws():
                    wait_store_bytes(sel)

                transpose_out(k, sel)
                fire_store(k, sel)

        wait_store_bytes(0)
        wait_store_bytes(1)

    return gather_rows


def kernel(indices, emb_table):
    bsz, hist = indices.shape
    vocab, dim = emb_table.shape
    assert dim == 64 and vocab % 128 == 64 and bsz % 128 == 0

    mesh = plsc.VectorSubcoreMesh(core_axis_name="c", subcore_axis_name="s")
    nc, ns = mesh.num_cores, mesh.num_subcores
    nw = nc * ns

    tview = emb_table.T
    n_full = vocab // 128
    tail = emb_table[n_full * 128:].reshape(32, 128)
    idx3 = indices.T.astype(jnp.int32).reshape(hist * (bsz // 128), 128)

    pack_table = _make_pack_kernel(vocab, dim, mesh, nw, nc)
    gather_rows = _make_gather_kernel(bsz, hist, dim, mesh, nw, nc)

    tpack = pack_table(tview, tail)
    out5 = gather_rows(tpack, idx3)
    return out5.transpose(2, 4, 0, 1, 3).reshape(bsz, hist, dim)


# speedup vs baseline: 1.7623x; 1.0235x over previous
import functools

import jax
import jax.numpy as jnp
from jax import lax
from jax.experimental import pallas as pl
from jax.experimental.pallas import tpu as pltpu
from jax.experimental.pallas import tpu_sc as plsc


def _iota16():
    return jnp.arange(16, dtype=jnp.int32)


def _make_pack_kernel(vocab, dim, mesh, nw, nc):
    n_full = vocab // 128
    tiles_per_w = (n_full + nw - 1) // nw

    @functools.partial(
        pl.kernel,
        out_type=jax.ShapeDtypeStruct((vocab // 2, 128), jnp.float32),
        mesh=mesh,
        scratch_types=[
            pltpu.VMEM((2, 64, 128), jnp.float32),
            pltpu.VMEM((2, 64, 128), jnp.float32),
            pltpu.VMEM((32, 128), jnp.float32),
            pltpu.SemaphoreType.DMA,
            pltpu.SemaphoreType.DMA,
            pltpu.SemaphoreType.DMA,
            pltpu.SemaphoreType.DMA,
        ],
        compiler_params=pltpu.CompilerParams(
            use_tc_tiling_on_sc=True, needs_layout_passes=False, disable_bounds_checks=True
        ),
    )
    def pack_table(tview_hbm, tail_hbm, tpack_hbm, ibuf, obuf, tailbuf, si0, si1, so0, so1):
        wid = lax.axis_index("s") * nc + lax.axis_index("c")
        n_valid = jnp.minimum(tiles_per_w, jnp.maximum(n_full - wid * tiles_per_w, 0))
        sem_i = (si0, si1)
        sem_o = (so0, so1)
        iota = _iota16()

        def fire_read(k, sel):
            t = wid * tiles_per_w + k
            for g in range(8):
                pltpu.async_copy(
                    tview_hbm.at[pl.ds(8 * g, 8), pl.ds(t * 128, 128)],
                    ibuf.at[sel, pl.ds(8 * g, 8)],
                    sem_i[sel],
                )

        def wait_read(k, sel):
            t = wid * tiles_per_w + k
            for g in range(8):
                pltpu.make_async_copy(
                    tview_hbm.at[pl.ds(8 * g, 8), pl.ds(t * 128, 128)],
                    ibuf.at[sel, pl.ds(8 * g, 8)],
                    sem_i[sel],
                ).wait()

        def fire_store(k, sel):
            t = wid * tiles_per_w + k
            pltpu.async_copy(
                obuf.at[sel],
                tpack_hbm.at[pl.ds(t * 64, 64)],
                sem_o[sel],
            )

        def wait_store_bytes(sel):
            pltpu.make_async_copy(
                obuf.at[sel],
                tpack_hbm.at[pl.ds(0, 64)],
                sem_o[sel],
            ).wait()

        def transpose_tile(sel):
            for jj in range(8):
                row = jnp.full((16,), (jj * 16) % 64, jnp.int32) + iota
                cb = jj // 4
                for q0 in range(0, 64, 16):
                    vecs = [
                        plsc.load_gather(
                            ibuf.at[sel],
                            [row, jnp.full((16,), 2 * (q0 + i) + cb, jnp.int32)],
                        )
                        for i in range(16)
                    ]
                    for i, vec in enumerate(vecs):
                        obuf[sel, q0 + i, pl.ds(jj * 16, 16)] = vec

        fire_read(0, 0)

        @pl.loop(0, tiles_per_w + 1, step=2)
        def _tile(k2):
            for sel in (0, 1):
                k = k2 + sel
                nxt = k + 1

                @pl.when(nxt < n_valid)
                def _fire():
                    fire_read(nxt, 1 - sel)

                @pl.when(k < n_valid)
                def _proc():
                    wait_read(k, sel)

                    @pl.when(k >= 2)
                    def _ws():
                        wait_store_bytes(sel)

                    transpose_tile(sel)
                    fire_store(k, sel)

        wait_store_bytes(0)
        wait_store_bytes(1)

        @pl.when(wid == nw - 1)
        def _tail():
            pltpu.sync_copy(tail_hbm, tailbuf)
            pltpu.sync_copy(tailbuf, tpack_hbm.at[pl.ds(n_full * 64, 32)])

    return pack_table


def _make_gather_kernel(bsz, hist, dim, mesh, nw, nc):
    n_items = hist * (bsz // 128)
    items_per_w = n_items // nw
    assert n_items == items_per_w * nw
    c_per_h = bsz // 128

    @functools.partial(
        pl.kernel,
        out_type=jax.ShapeDtypeStruct((hist, 8, c_per_h, 8, 128), jnp.float32),
        mesh=mesh,
        scratch_types=[
            pltpu.VMEM((items_per_w, 128), jnp.int32),
            pltpu.VMEM((2, 128), jnp.int32),
            pltpu.VMEM((2, 128, 128), jnp.float32),
            pltpu.VMEM((2, 8, 8, 128), jnp.float32),
            pltpu.SemaphoreType.DMA,
            pltpu.SemaphoreType.DMA,
            pltpu.SemaphoreType.DMA,
            pltpu.SemaphoreType.DMA,
        ],
        compiler_params=pltpu.CompilerParams(
            use_tc_tiling_on_sc=False, needs_layout_passes=False, disable_bounds_checks=True
        ),
    )
    def gather_rows(tpack_hbm, idx3_hbm, out5_hbm, idx_all, prow, grows, obuf,
                    sg0, sg1, so0, so1):
        wid = lax.axis_index("s") * nc + lax.axis_index("c")
        sem_g = (sg0, sg1)
        sem_o = (so0, so1)
        iota = _iota16()

        pltpu.sync_copy(idx3_hbm.at[pl.ds(wid * items_per_w, items_per_w)], idx_all)

        def fire_gather(k, sel):
            for lc in range(8):
                v = idx_all[k, pl.ds(lc * 16, 16)]
                prow[sel, pl.ds(lc * 16, 16)] = v >> 1
            pltpu.async_copy(tpack_hbm.at[prow.at[sel]], grows.at[sel], sem_g[sel])

        def wait_gather(sel):
            pltpu.make_async_copy(
                tpack_hbm.at[prow.at[sel]], grows.at[sel], sem_g[sel]
            ).wait()

        def transpose_out(k, sel):
            for lc in range(8):
                v = idx_all[k, pl.ds(lc * 16, 16)]
                hoff = (v & 1) << 6
                row = jnp.full((16,), lc * 16, jnp.int32) + iota
                for d0 in range(0, 64, 16):
                    vecs = [
                        plsc.load_gather(grows.at[sel], [row, hoff + (d0 + i)])
                        for i in range(16)
                    ]
                    for i, vec in enumerate(vecs):
                        d = d0 + i
                        obuf[sel, d // 8, d % 8, pl.ds(lc * 16, 16)] = vec

        def fire_store(k, sel):
            m = wid * items_per_w + k
            h = m // c_per_h
            c = m % c_per_h
            pltpu.async_copy(obuf.at[sel], out5_hbm.at[h, :, c], sem_o[sel])

        def wait_store_bytes(sel):
            pltpu.make_async_copy(
                obuf.at[sel], out5_hbm.at[0, :, 0], sem_o[sel]
            ).wait()

        fire_gather(0, 0)

        @pl.loop(0, items_per_w, step=2)
        def _item(k2):
            for sel in (0, 1):
                k = k2 + sel
                nxt = k + 1

                @pl.when(nxt < items_per_w)
                def _fire():
                    fire_gather(nxt, 1 - sel)

                wait_gather(sel)

                @pl.when(k >= 2)
                def _ws():
                    wait_store_bytes(sel)

                transpose_out(k, sel)
                fire_store(k, sel)

        wait_store_bytes(0)
        wait_store_bytes(1)

    return gather_rows


def kernel(indices, emb_table):
    bsz, hist = indices.shape
    vocab, dim = emb_table.shape
    assert dim == 64 and vocab % 128 == 64 and bsz % 128 == 0

    mesh = plsc.VectorSubcoreMesh(core_axis_name="c", subcore_axis_name="s")
    nc, ns = mesh.num_cores, mesh.num_subcores
    nw = nc * ns

    tview = emb_table.T
    n_full = vocab // 128
    tail = emb_table[n_full * 128:].reshape(32, 128)
    idx3 = indices.T.astype(jnp.int32).reshape(hist * (bsz // 128), 128)

    pack_table = _make_pack_kernel(vocab, dim, mesh, nw, nc)
    gather_rows = _make_gather_kernel(bsz, hist, dim, mesh, nw, nc)

    tpack = pack_table(tview, tail)
    out5 = gather_rows(tpack, idx3)
    return out5.transpose(2, 4, 0, 1, 3).reshape(bsz, hist, dim)


# speedup vs baseline: 1.7667x; 1.0025x over previous
import functools

import jax
import jax.numpy as jnp
from jax import lax
from jax.experimental import pallas as pl
from jax.experimental.pallas import tpu as pltpu
from jax.experimental.pallas import tpu_sc as plsc


def _iota16():
    return jnp.arange(16, dtype=jnp.int32)


def _make_pack_kernel(vocab, dim, mesh, nw, nc):
    n_full = vocab // 128
    tiles_per_w = (n_full + nw - 1) // nw

    @functools.partial(
        pl.kernel,
        out_type=jax.ShapeDtypeStruct((vocab // 2, 128), jnp.float32),
        mesh=mesh,
        scratch_types=[
            pltpu.VMEM((2, 64, 128), jnp.float32),
            pltpu.VMEM((2, 64, 128), jnp.float32),
            pltpu.VMEM((32, 128), jnp.float32),
            pltpu.SemaphoreType.DMA,
            pltpu.SemaphoreType.DMA,
            pltpu.SemaphoreType.DMA,
            pltpu.SemaphoreType.DMA,
        ],
        compiler_params=pltpu.CompilerParams(
            use_tc_tiling_on_sc=True, needs_layout_passes=False, disable_bounds_checks=True
        ),
    )
    def pack_table(tview_hbm, tail_hbm, tpack_hbm, ibuf, obuf, tailbuf, si0, si1, so0, so1):
        wid = lax.axis_index("s") * nc + lax.axis_index("c")
        n_valid = jnp.minimum(tiles_per_w, jnp.maximum(n_full - wid * tiles_per_w, 0))
        sem_i = (si0, si1)
        sem_o = (so0, so1)
        iota = _iota16()

        def fire_read(k, sel):
            t = wid * tiles_per_w + k
            for g in range(8):
                pltpu.async_copy(
                    tview_hbm.at[pl.ds(8 * g, 8), pl.ds(t * 128, 128)],
                    ibuf.at[sel, pl.ds(8 * g, 8)],
                    sem_i[sel],
                )

        def wait_read(k, sel):
            t = wid * tiles_per_w + k
            for g in range(8):
                pltpu.make_async_copy(
                    tview_hbm.at[pl.ds(8 * g, 8), pl.ds(t * 128, 128)],
                    ibuf.at[sel, pl.ds(8 * g, 8)],
                    sem_i[sel],
                ).wait()

        def fire_store(k, sel):
            t = wid * tiles_per_w + k
            pltpu.async_copy(
                obuf.at[sel],
                tpack_hbm.at[pl.ds(t * 64, 64)],
                sem_o[sel],
            )

        def wait_store_bytes(sel):
            pltpu.make_async_copy(
                obuf.at[sel],
                tpack_hbm.at[pl.ds(0, 64)],
                sem_o[sel],
            ).wait()

        def transpose_tile(sel):
            for jj in range(8):
                row = jnp.full((16,), (jj * 16) % 64, jnp.int32) + iota
                cb = jj // 4
                for q0 in range(0, 64, 16):
                    vecs = [
                        plsc.load_gather(
                            ibuf.at[sel],
                            [row, jnp.full((16,), 2 * (q0 + i) + cb, jnp.int32)],
                        )
                        for i in range(16)
                    ]
                    for i, vec in enumerate(vecs):
                        obuf[sel, q0 + i, pl.ds(jj * 16, 16)] = vec

        fire_read(0, 0)

        @pl.loop(0, tiles_per_w + 1, step=2)
        def _tile(k2):
            for sel in (0, 1):
                k = k2 + sel
                nxt = k + 1

                @pl.when(nxt < n_valid)
                def _fire():
                    fire_read(nxt, 1 - sel)

                @pl.when(k < n_valid)
                def _proc():
                    wait_read(k, sel)

                    @pl.when(k >= 2)
                    def _ws():
                        wait_store_bytes(sel)

                    transpose_tile(sel)
                    fire_store(k, sel)

        wait_store_bytes(0)
        wait_store_bytes(1)

        @pl.when(wid == nw - 1)
        def _tail():
            pltpu.sync_copy(tail_hbm, tailbuf)
            pltpu.sync_copy(tailbuf, tpack_hbm.at[pl.ds(n_full * 64, 32)])

    return pack_table


def _make_gather_kernel(bsz, hist, dim, mesh, nw, nc):
    n_items = hist * (bsz // 128)
    items_per_w = n_items // nw
    assert n_items == items_per_w * nw
    c_per_h = bsz // 128

    @functools.partial(
        pl.kernel,
        out_type=jax.ShapeDtypeStruct((hist, 8, c_per_h, 8, 128), jnp.float32),
        mesh=mesh,
        scratch_types=[
            pltpu.VMEM((items_per_w, 128), jnp.int32),
            pltpu.VMEM((2, 128, 64), jnp.float32),
            pltpu.VMEM((2, 8, 8, 128), jnp.float32),
            pltpu.SemaphoreType.DMA,
            pltpu.SemaphoreType.DMA,
            pltpu.SemaphoreType.DMA,
            pltpu.SemaphoreType.DMA,
        ],
        compiler_params=pltpu.CompilerParams(
            use_tc_tiling_on_sc=False, needs_layout_passes=False, disable_bounds_checks=True
        ),
    )
    def gather_rows(tpack_hbm, idx3_hbm, out5_hbm, idx_all, grows, obuf,
                    sg0, sg1, so0, so1):
        wid = lax.axis_index("s") * nc + lax.axis_index("c")
        sem_g = (sg0, sg1)
        sem_o = (so0, so1)
        iota = _iota16()

        pltpu.sync_copy(idx3_hbm.at[pl.ds(wid * items_per_w, items_per_w)], idx_all)

        def fire_gather(k, sel):
            pltpu.async_copy(tpack_hbm.at[idx_all.at[k]], grows.at[sel], sem_g[sel])

        def wait_gather(k, sel):
            pltpu.make_async_copy(
                tpack_hbm.at[idx_all.at[k]], grows.at[sel], sem_g[sel]
            ).wait()

        def transpose_out(k, sel):
            for lc in range(8):
                row = jnp.full((16,), lc * 16, jnp.int32) + iota
                for d0 in range(0, 64, 16):
                    vecs = [
                        plsc.load_gather(
                            grows.at[sel],
                            [row, jnp.full((16,), d0 + i, jnp.int32)],
                        )
                        for i in range(16)
                    ]
                    for i, vec in enumerate(vecs):
                        d = d0 + i
                        obuf[sel, d // 8, d % 8, pl.ds(lc * 16, 16)] = vec

        def fire_store(k, sel):
            m = wid * items_per_w + k
            h = m // c_per_h
            c = m % c_per_h
            pltpu.async_copy(obuf.at[sel], out5_hbm.at[h, :, c], sem_o[sel])

        def wait_store_bytes(sel):
            pltpu.make_async_copy(
                obuf.at[sel], out5_hbm.at[0, :, 0], sem_o[sel]
            ).wait()

        fire_gather(0, 0)

        @pl.loop(0, items_per_w, step=2)
        def _item(k2):
            for sel in (0, 1):
                k = k2 + sel
                nxt = k + 1

                @pl.when(nxt < items_per_w)
                def _fire():
                    fire_gather(nxt, 1 - sel)

                wait_gather(k, sel)

                @pl.when(k >= 2)
                def _ws():
                    wait_store_bytes(sel)

                transpose_out(k, sel)
                fire_store(k, sel)

        wait_store_bytes(0)
        wait_store_bytes(1)

    return gather_rows


def kernel(indices, emb_table):
    bsz, hist = indices.shape
    vocab, dim = emb_table.shape
    assert dim == 64 and vocab % 128 == 64 and bsz % 128 == 0

    mesh = plsc.VectorSubcoreMesh(core_axis_name="c", subcore_axis_name="s")
    nc, ns = mesh.num_cores, mesh.num_subcores
    nw = nc * ns

    tview = emb_table.T
    n_full = vocab // 128
    tail = emb_table[n_full * 128:].reshape(32, 128)
    idx3 = indices.T.astype(jnp.int32).reshape(hist * (bsz // 128), 128)

    pack_table = _make_pack_kernel(vocab, dim, mesh, nw, nc)
    gather_rows = _make_gather_kernel(bsz, hist, dim, mesh, nw, nc)

    tpack = pack_table(tview, tail)
    out5 = gather_rows(tpack.reshape(vocab, dim), idx3)
    return out5.transpose(2, 4, 0, 1, 3).reshape(bsz, hist, dim)


# speedup vs baseline: 1.9070x; 1.0794x over previous
import functools

import jax
import jax.numpy as jnp
from jax import lax
from jax.experimental import pallas as pl
from jax.experimental.pallas import tpu as pltpu
from jax.experimental.pallas import tpu_sc as plsc


def _iota16():
    return jnp.arange(16, dtype=jnp.int32)


def _make_pack_kernel(vocab, dim, mesh, nw, nc):
    n_full = vocab // 128
    tiles_per_w = (n_full + nw - 1) // nw

    @functools.partial(
        pl.kernel,
        out_type=jax.ShapeDtypeStruct((vocab // 2, 128), jnp.float32),
        mesh=mesh,
        scratch_types=[
            pltpu.VMEM((2, 64, 128), jnp.float32),
            pltpu.VMEM((2, 64, 128), jnp.float32),
            pltpu.VMEM((32, 128), jnp.float32),
            pltpu.SemaphoreType.DMA,
            pltpu.SemaphoreType.DMA,
            pltpu.SemaphoreType.DMA,
            pltpu.SemaphoreType.DMA,
        ],
        compiler_params=pltpu.CompilerParams(
            use_tc_tiling_on_sc=True, needs_layout_passes=False, disable_bounds_checks=True
        ),
    )
    def pack_table(tview_hbm, tail_hbm, tpack_hbm, ibuf, obuf, tailbuf, si0, si1, so0, so1):
        wid = lax.axis_index("s") * nc + lax.axis_index("c")
        n_valid = jnp.minimum(tiles_per_w, jnp.maximum(n_full - wid * tiles_per_w, 0))
        sem_i = (si0, si1)
        sem_o = (so0, so1)
        iota = _iota16()

        def fire_read(k, sel):
            t = wid * tiles_per_w + k
            for g in range(8):
                pltpu.async_copy(
                    tview_hbm.at[pl.ds(8 * g, 8), pl.ds(t * 128, 128)],
                    ibuf.at[sel, pl.ds(8 * g, 8)],
                    sem_i[sel],
                )

        def wait_read(k, sel):
            t = wid * tiles_per_w + k
            for g in range(8):
                pltpu.make_async_copy(
                    tview_hbm.at[pl.ds(8 * g, 8), pl.ds(t * 128, 128)],
                    ibuf.at[sel, pl.ds(8 * g, 8)],
                    sem_i[sel],
                ).wait()

        def fire_store(k, sel):
            t = wid * tiles_per_w + k
            pltpu.async_copy(
                obuf.at[sel],
                tpack_hbm.at[pl.ds(t * 64, 64)],
                sem_o[sel],
            )

        def wait_store_bytes(sel):
            pltpu.make_async_copy(
                obuf.at[sel],
                tpack_hbm.at[pl.ds(0, 64)],
                sem_o[sel],
            ).wait()

        def transpose_tile(sel):
            for jj in range(8):
                row = jnp.full((16,), (jj * 16) % 64, jnp.int32) + iota
                cb = jj // 4

                @plsc.parallel_loop(0, 64, step=1, unroll=8)
                def _q(q, _jj=jj, _row=row, _cb=cb, _sel=sel):
                    col = jnp.full((16,), _cb, jnp.int32) + 2 * q
                    vec = plsc.load_gather(ibuf.at[_sel], [_row, col])
                    obuf[_sel, q, pl.ds(_jj * 16, 16)] = vec

        fire_read(0, 0)

        @pl.loop(0, tiles_per_w + 1, step=2)
        def _tile(k2):
            for sel in (0, 1):
                k = k2 + sel
                nxt = k + 1

                @pl.when(nxt < n_valid)
                def _fire():
                    fire_read(nxt, 1 - sel)

                @pl.when(k < n_valid)
                def _proc():
                    wait_read(k, sel)

                    @pl.when(k >= 2)
                    def _ws():
                        wait_store_bytes(sel)

                    transpose_tile(sel)
                    fire_store(k, sel)

        wait_store_bytes(0)
        wait_store_bytes(1)

        @pl.when(wid == nw - 1)
        def _tail():
            pltpu.sync_copy(tail_hbm, tailbuf)
            pltpu.sync_copy(tailbuf, tpack_hbm.at[pl.ds(n_full * 64, 32)])

    return pack_table


def _make_gather_kernel(bsz, hist, dim, mesh, nw, nc):
    n_items = hist * (bsz // 128)
    items_per_w = n_items // nw
    assert n_items == items_per_w * nw
    c_per_h = bsz // 128

    @functools.partial(
        pl.kernel,
        out_type=jax.ShapeDtypeStruct((hist, 8, c_per_h, 8, 128), jnp.float32),
        mesh=mesh,
        scratch_types=[
            pltpu.VMEM((items_per_w, 128), jnp.int32),
            pltpu.VMEM((2, 128, 64), jnp.float32),
            pltpu.VMEM((2, 8, 8, 128), jnp.float32),
            pltpu.SemaphoreType.DMA,
            pltpu.SemaphoreType.DMA,
            pltpu.SemaphoreType.DMA,
            pltpu.SemaphoreType.DMA,
        ],
        compiler_params=pltpu.CompilerParams(
            use_tc_tiling_on_sc=False, needs_layout_passes=False, disable_bounds_checks=True
        ),
    )
    def gather_rows(tpack_hbm, idx3_hbm, out5_hbm, idx_all, grows, obuf,
                    sg0, sg1, so0, so1):
        wid = lax.axis_index("s") * nc + lax.axis_index("c")
        sem_g = (sg0, sg1)
        sem_o = (so0, so1)
        iota = _iota16()

        pltpu.sync_copy(idx3_hbm.at[pl.ds(wid * items_per_w, items_per_w)], idx_all)

        def fire_gather(k, sel):
            pltpu.async_copy(tpack_hbm.at[idx_all.at[k]], grows.at[sel], sem_g[sel])

        def wait_gather(k, sel):
            pltpu.make_async_copy(
                tpack_hbm.at[idx_all.at[k]], grows.at[sel], sem_g[sel]
            ).wait()

        def transpose_out(k, sel):
            for lc in range(8):
                row = jnp.full((16,), lc * 16, jnp.int32) + iota

                @plsc.parallel_loop(0, 64, step=1, unroll=8)
                def _d(d, _lc=lc, _row=row, _sel=sel):
                    col = jnp.full((16,), 0, jnp.int32) + d
                    vec = plsc.load_gather(grows.at[_sel], [_row, col])
                    obuf[_sel, d // 8, d % 8, pl.ds(_lc * 16, 16)] = vec

        def fire_store(k, sel):
            m = wid * items_per_w + k
            h = m // c_per_h
            c = m % c_per_h
            pltpu.async_copy(obuf.at[sel], out5_hbm.at[h, :, c], sem_o[sel])

        def wait_store_bytes(sel):
            pltpu.make_async_copy(
                obuf.at[sel], out5_hbm.at[0, :, 0], sem_o[sel]
            ).wait()

        fire_gather(0, 0)

        @pl.loop(0, items_per_w, step=2)
        def _item(k2):
            for sel in (0, 1):
                k = k2 + sel
                nxt = k + 1

                @pl.when(nxt < items_per_w)
                def _fire():
                    fire_gather(nxt, 1 - sel)

                wait_gather(k, sel)

                @pl.when(k >= 2)
                def _ws():
                    wait_store_bytes(sel)

                transpose_out(k, sel)
                fire_store(k, sel)

        wait_store_bytes(0)
        wait_store_bytes(1)

    return gather_rows


def kernel(indices, emb_table):
    bsz, hist = indices.shape
    vocab, dim = emb_table.shape
    assert dim == 64 and vocab % 128 == 64 and bsz % 128 == 0

    mesh = plsc.VectorSubcoreMesh(core_axis_name="c", subcore_axis_name="s")
    nc, ns = mesh.num_cores, mesh.num_subcores
    nw = nc * ns

    tview = emb_table.T
    n_full = vocab // 128
    tail = emb_table[n_full * 128:].reshape(32, 128)
    idx3 = indices.T.astype(jnp.int32).reshape(hist * (bsz // 128), 128)

    pack_table = _make_pack_kernel(vocab, dim, mesh, nw, nc)
    gather_rows = _make_gather_kernel(bsz, hist, dim, mesh, nw, nc)

    tpack = pack_table(tview, tail)
    out5 = gather_rows(tpack.reshape(vocab, dim), idx3)
    return out5.transpose(2, 4, 0, 1, 3).reshape(bsz, hist, dim)


# speedup vs baseline: 2.2934x; 1.2026x over previous
import functools

import jax
import jax.numpy as jnp
from jax import lax
from jax.experimental import pallas as pl
from jax.experimental.pallas import tpu as pltpu
from jax.experimental.pallas import tpu_sc as plsc

_CHUNK = 128
_K = 4


def kernel(indices, emb_table):
    bsz, hist = indices.shape
    vocab, dim = emb_table.shape
    n = bsz * hist

    mesh = plsc.VectorSubcoreMesh(core_axis_name="c", subcore_axis_name="s")
    nc, ns = mesh.num_cores, mesh.num_subcores
    nw = nc * ns
    n_chunks = n // (nw * _CHUNK)
    assert n == nw * n_chunks * _CHUNK, (n, nw, n_chunks)
    n_groups = n_chunks // _K
    assert n_chunks == n_groups * _K and n_groups % 2 == 0, (n_chunks, n_groups)
    rows_per_w = n_chunks * _CHUNK
    grp_rows = _K * _CHUNK

    idx3 = indices.reshape(nw, n_chunks, _CHUNK).astype(jnp.int32)

    @functools.partial(
        pl.kernel,
        out_type=jax.ShapeDtypeStruct((nw, rows_per_w, dim), jnp.float32),
        mesh=mesh,
        scratch_types=[
            pltpu.VMEM((n_chunks, _CHUNK), jnp.int32),
            pltpu.VMEM((2, grp_rows, dim), jnp.float32),
            pltpu.SemaphoreType.DMA,
            pltpu.SemaphoreType.DMA,
            pltpu.SemaphoreType.DMA,
            pltpu.SemaphoreType.DMA,
        ],
        compiler_params=pltpu.CompilerParams(use_tc_tiling_on_sc=False),
    )
    def emb_gather(table_hbm, idx_hbm, out_hbm, idx_v, rows_v, sg0, sg1, ss0, ss1):
        wid = lax.axis_index("s") * nc + lax.axis_index("c")
        sem_g = (sg0, sg1)
        sem_s = (ss0, ss1)
        pltpu.sync_copy(idx_hbm.at[wid], idx_v)

        def fire_gathers(grp, half):
            for b in range(_K):
                c = grp * _K + b
                pltpu.async_copy(
                    table_hbm.at[idx_v.at[c]],
                    rows_v.at[half, pl.ds(b * _CHUNK, _CHUNK)],
                    sem_g[half],
                )

        def drain_gathers(grp, half):
            for b in range(_K):
                c = grp * _K + b
                pltpu.make_async_copy(
                    table_hbm.at[idx_v.at[c]],
                    rows_v.at[half, pl.ds(b * _CHUNK, _CHUNK)],
                    sem_g[half],
                ).wait()

        def start_store(grp, half):
            pltpu.async_copy(
                rows_v.at[half],
                out_hbm.at[wid, pl.ds(grp * grp_rows, grp_rows)],
                sem_s[half],
            )

        def wait_store(grp, half):
            pltpu.make_async_copy(
                rows_v.at[half],
                out_hbm.at[wid, pl.ds(grp * grp_rows, grp_rows)],
                sem_s[half],
            ).wait()

        fire_gathers(0, 0)

        @pl.loop(0, n_groups, step=2)
        def _grp(g):
            for sel in (0, 1):
                gg = g + sel
                nxt = gg + 1
                other = 1 - sel

                @pl.when(nxt < n_groups)
                def _fire_next():
                    @pl.when(nxt >= 2)
                    def _wait_prev_store():
                        wait_store(nxt - 2, other)

                    fire_gathers(nxt, other)

                drain_gathers(gg, sel)
                start_store(gg, sel)

        wait_store(n_groups - 2, 0)
        wait_store(n_groups - 1, 1)

    out = emb_gather(emb_table, idx3)
    return out.reshape(bsz, hist, dim)
